# baseline (speedup 1.0000x reference)
import functools

import jax
import jax.numpy as jnp
from jax.experimental import pallas as pl
from jax.experimental.pallas import tpu as pltpu

NUM_EXPERTS = 64
TOP_K = 2
BT = 512
NBUF = 8
NPRI = 2


def _top2(scores):
    i1 = jnp.argmax(scores, axis=-1)
    m1 = jnp.max(scores, axis=-1)
    cols = jax.lax.broadcasted_iota(jnp.int32, scores.shape, 1)
    masked = jnp.where(cols == i1[:, None], -jnp.inf, scores)
    i2 = jnp.argmax(masked, axis=-1)
    m2 = jnp.max(masked, axis=-1)
    return (jnp.stack([m1, m2], axis=-1),
            jnp.stack([i1, i2], axis=-1).astype(jnp.int32))


def _router_kernel(x_hbm, wt_ref, w_out_ref, i_out_ref, s_out_ref,
                   buf_ref, sem_ref):
    n_tokens = x_hbm.shape[0]
    nchunk = n_tokens // BT
    wt = wt_ref[...]

    def start_copy(c):
        slot = c % NBUF
        pltpu.make_async_copy(
            x_hbm.at[pl.ds(c * BT, BT), :],
            buf_ref.at[slot],
            sem_ref.at[slot],
        ).start(priority=c % NPRI)

    for c in range(min(NBUF, nchunk)):
        start_copy(c)

    for c in range(nchunk):
        slot = c % NBUF
        pltpu.make_async_copy(
            x_hbm.at[pl.ds(c * BT, BT), :],
            buf_ref.at[slot],
            sem_ref.at[slot],
        ).wait()
        xb = buf_ref[slot]
        logits = jnp.dot(xb, wt, preferred_element_type=jnp.float32)
        m = jnp.max(logits, axis=-1, keepdims=True)
        e = jnp.exp(logits - m)
        s = jnp.sum(e, axis=-1, keepdims=True)
        scores = e / s
        s_out_ref[pl.ds(c * BT, BT), :] = scores
        w, i = _top2(scores)
        w_out_ref[pl.ds(c * BT, BT), :] = w
        i_out_ref[pl.ds(c * BT, BT), :] = i
        nxt = c + NBUF
        if nxt < nchunk:
            start_copy(nxt)


@functools.partial(jax.jit, static_argnames=())
def kernel(x, W):
    n_tokens, d_model = x.shape
    wt = W.T
    weights, indices, scores = pl.pallas_call(
        _router_kernel,
        in_specs=[
            pl.BlockSpec(memory_space=pltpu.HBM),
            pl.BlockSpec(memory_space=pltpu.VMEM),
        ],
        out_specs=[
            pl.BlockSpec(memory_space=pltpu.VMEM),
            pl.BlockSpec(memory_space=pltpu.VMEM),
            pl.BlockSpec(memory_space=pltpu.VMEM),
        ],
        out_shape=[
            jax.ShapeDtypeStruct((n_tokens, TOP_K), jnp.float32),
            jax.ShapeDtypeStruct((n_tokens, TOP_K), jnp.int32),
            jax.ShapeDtypeStruct((n_tokens, NUM_EXPERTS), jnp.float32),
        ],
        scratch_shapes=[
            pltpu.VMEM((NBUF, BT, d_model), jnp.float32),
            pltpu.SemaphoreType.DMA((NBUF,)),
        ],
    )(x, wt)
    return weights, indices, scores


# speedup vs baseline: 1.1067x; 1.1067x over previous
import functools

import jax
import jax.numpy as jnp
from jax.experimental import pallas as pl
from jax.experimental.pallas import tpu as pltpu

NUM_EXPERTS = 64
TOP_K = 2
BT = 512
NBUF = 8
NPRI = 1


def _top2(scores):
    i1 = jnp.argmax(scores, axis=-1)
    m1 = jnp.max(scores, axis=-1)
    cols = jax.lax.broadcasted_iota(jnp.int32, scores.shape, 1)
    masked = jnp.where(cols == i1[:, None], -jnp.inf, scores)
    i2 = jnp.argmax(masked, axis=-1)
    m2 = jnp.max(masked, axis=-1)
    return (jnp.stack([m1, m2], axis=-1),
            jnp.stack([i1, i2], axis=-1).astype(jnp.int32))


def _router_kernel(x_hbm, wt_ref, w_out_ref, i_out_ref, s_out_ref,
                   buf_ref, sem_ref):
    n_tokens = x_hbm.shape[0]
    nchunk = n_tokens // BT
    wt = wt_ref[...]

    def start_copy(c):
        slot = c % NBUF
        pltpu.make_async_copy(
            x_hbm.at[pl.ds(c * BT, BT), :],
            buf_ref.at[slot],
            sem_ref.at[slot],
        ).start(priority=c % NPRI)

    for c in range(min(NBUF, nchunk)):
        start_copy(c)

    for c in range(nchunk):
        slot = c % NBUF
        pltpu.make_async_copy(
            x_hbm.at[pl.ds(c * BT, BT), :],
            buf_ref.at[slot],
            sem_ref.at[slot],
        ).wait()
        xb = buf_ref[slot]
        logits = jnp.dot(xb, wt, preferred_element_type=jnp.float32)
        m = jnp.max(logits, axis=-1, keepdims=True)
        e = jnp.exp(logits - m)
        s = jnp.sum(e, axis=-1, keepdims=True)
        scores = e / s
        s_out_ref[pl.ds(c * BT, BT), :] = scores
        w, i = _top2(scores)
        w_out_ref[pl.ds(c * BT, BT), :] = w
        i_out_ref[pl.ds(c * BT, BT), :] = i
        nxt = c + NBUF
        if nxt < nchunk:
            start_copy(nxt)


@functools.partial(jax.jit, static_argnames=())
def kernel(x, W):
    n_tokens, d_model = x.shape
    wt = W.T
    weights, indices, scores = pl.pallas_call(
        _router_kernel,
        in_specs=[
            pl.BlockSpec(memory_space=pltpu.HBM),
            pl.BlockSpec(memory_space=pltpu.VMEM),
        ],
        out_specs=[
            pl.BlockSpec(memory_space=pltpu.VMEM),
            pl.BlockSpec(memory_space=pltpu.VMEM),
            pl.BlockSpec(memory_space=pltpu.VMEM),
        ],
        out_shape=[
            jax.ShapeDtypeStruct((n_tokens, TOP_K), jnp.float32),
            jax.ShapeDtypeStruct((n_tokens, TOP_K), jnp.int32),
            jax.ShapeDtypeStruct((n_tokens, NUM_EXPERTS), jnp.float32),
        ],
        scratch_shapes=[
            pltpu.VMEM((NBUF, BT, d_model), jnp.float32),
            pltpu.SemaphoreType.DMA((NBUF,)),
        ],
    )(x, wt)
    return weights, indices, scores
